# Initial kernel scaffold; baseline (speedup 1.0000x reference)
import jax
import jax.numpy as jnp
from jax.experimental import pallas as pl


def kernel(x, ei, pos, W1, b1, W2, b2, Wm1, bm1, g1, be1, Wm2, bm2, g2, be2, W3a, b3a, W3b, b3b):
    raise NotImplementedError("write your pallas kernel here")


# speedup vs baseline: 1.3037x; 1.3037x over previous
import functools

import jax
import jax.numpy as jnp
from jax.experimental import pallas as pl
from jax.experimental.pallas import tpu as pltpu

N = 1024
E = 32768
P = 768
F = 128
H = 32


def _gcn_body(x_ref, ahat_ref, w1_ref, b1_ref, w2_ref, b2_ref, h_ref):
    hp = jax.lax.Precision.HIGHEST
    xw1 = jax.lax.dot(x_ref[...], w1_ref[...])
    h1 = jax.lax.dot(ahat_ref[...], xw1, precision=hp) + b1_ref[...]
    h1w2 = jax.lax.dot(h1, w2_ref[...])
    h_ref[...] = jax.lax.dot(ahat_ref[...], h1w2, precision=hp) + b2_ref[...]


def _tc_gcn(x, ahat, W1, b1, W2, b2):
    return pl.pallas_call(
        _gcn_body,
        out_shape=jax.ShapeDtypeStruct((N, H), jnp.float32),
    )(x, ahat, W1, b1.reshape(1, H), W2, b2.reshape(1, H))


def _edge_mlp_body(xe_ref, wm1_ref, bm1_ref, g1_ref, be1_ref,
                   wm2_ref, bm2_ref, g2_ref, be2_ref, x1_ref, x2_ref):
    xe = xe_ref[...]

    def ln_relu(z, g, b):
        mu = jnp.mean(z, axis=-1, keepdims=True)
        var = jnp.mean((z - mu) ** 2, axis=-1, keepdims=True)
        return jax.nn.relu((z - mu) * jax.lax.rsqrt(var + 1e-5) * g + b)

    z1 = jax.lax.dot(xe, wm1_ref[...]) + bm1_ref[...]
    x1_ref[...] = ln_relu(z1, g1_ref[...], be1_ref[...])
    z2 = jax.lax.dot(xe, wm2_ref[...]) + bm2_ref[...]
    x2_ref[...] = ln_relu(z2, g2_ref[...], be2_ref[...])


def _tc_edge_mlps(xe, Wm1, bm1, g1, be1, Wm2, bm2, g2, be2):
    blk = 8192
    grid = E // blk
    row_spec = pl.BlockSpec((blk, H), lambda i: (i, 0))
    w_spec = pl.BlockSpec((H, H), lambda i: (0, 0))
    v_spec = pl.BlockSpec((1, H), lambda i: (0, 0))
    return pl.pallas_call(
        _edge_mlp_body,
        grid=(grid,),
        in_specs=[row_spec, w_spec, v_spec, v_spec, v_spec,
                  w_spec, v_spec, v_spec, v_spec],
        out_specs=[row_spec, row_spec],
        out_shape=[jax.ShapeDtypeStruct((E, H), jnp.float32),
                   jax.ShapeDtypeStruct((E, H), jnp.float32)],
    )(xe, Wm1, bm1.reshape(1, H), g1.reshape(1, H), be1.reshape(1, H),
      Wm2, bm2.reshape(1, H), g2.reshape(1, H), be2.reshape(1, H))


def _final_body(pv_ref, xx_ref, w3a_ref, b3a_ref, w3b_ref, b3b_ref, out_ref):
    w3a = w3a_ref[...]
    h3 = jax.nn.relu(
        jax.lax.dot(pv_ref[...], w3a[:H, :])
        + jax.lax.dot(xx_ref[...], w3a[H:, :])
        + b3a_ref[...])
    out_ref[...] = jax.lax.dot(h3, w3b_ref[...]) + b3b_ref[...]


def _tc_final(pos_val, xx, W3a, b3a, W3b, b3b):
    out2 = pl.pallas_call(
        _final_body,
        out_shape=jax.ShapeDtypeStruct((P, 1), jnp.float32),
    )(pos_val, xx, W3a, b3a.reshape(1, H), W3b, b3b.reshape(1, 1))
    return out2[:, 0]


def kernel(x, ei, pos, W1, b1, W2, b2, Wm1, bm1, g1, be1, Wm2, bm2, g2, be2,
           W3a, b3a, W3b, b3b):
    src, dst = ei[0], ei[1]

    loop = jnp.arange(N)
    s_all = jnp.concatenate([src, loop])
    d_all = jnp.concatenate([dst, loop])
    deg = jnp.zeros((N,), jnp.float32).at[d_all].add(1.0)
    dinv = jnp.where(deg > 0, deg ** -0.5, 0.0)
    norm = dinv[s_all] * dinv[d_all]
    ahat = jnp.zeros((N, N), jnp.float32).at[d_all, s_all].add(norm)

    h = _tc_gcn(x, ahat, W1, b1, W2, b2)

    xx = h[pos[0]] * h[pos[1]]
    xe = h[src] * h[dst]

    x1, x2 = _tc_edge_mlps(xe, Wm1, bm1, g1, be1, Wm2, bm2, g2, be2)

    eid = jnp.arange(E, dtype=jnp.int32) + 1
    Lr = jnp.zeros((N, N), jnp.int32).at[src, dst].set(eid)
    Lc = jnp.zeros((N, N), jnp.int32).at[dst, src].set(eid)
    rown = Lr[pos[0]]
    coln = Lc[pos[1]]
    valid = (rown > 0) & (coln > 0)
    v2 = jnp.where(valid[:, :, None], x2[jnp.maximum(rown - 1, 0)], 0.0)
    v1 = jnp.where(valid[:, :, None], x1[jnp.maximum(coln - 1, 0)], 0.0)
    pos_val = jnp.sum(v2 * v1, axis=1)

    return _tc_final(pos_val, xx, W3a, b3a, W3b, b3b)


# speedup vs baseline: 7.1432x; 5.4790x over previous
import functools

import jax
import jax.numpy as jnp
from jax import lax
from jax.experimental import pallas as pl
from jax.experimental.pallas import tpu as pltpu
from jax.experimental.pallas import tpu_sc as plsc

N = 1024
E = 32768
P = 768
F = 128
H = 32

NW = 32
PP = P // NW
NSL = N // 16
WLCAP = N + 16


def _gcn_body(x_ref, ahat_ref, w1_ref, b1_ref, w2_ref, b2_ref, h_ref):
    hp = jax.lax.Precision.HIGHEST
    xw1 = jax.lax.dot(x_ref[...], w1_ref[...])
    h1 = jax.lax.dot(ahat_ref[...], xw1, precision=hp) + b1_ref[...]
    h1w2 = jax.lax.dot(h1, w2_ref[...])
    h_ref[...] = jax.lax.dot(ahat_ref[...], h1w2, precision=hp) + b2_ref[...]


def _tc_gcn(x, ahat, W1, b1, W2, b2):
    return pl.pallas_call(
        _gcn_body,
        out_shape=jax.ShapeDtypeStruct((N, H), jnp.float32),
    )(x, ahat, W1, b1.reshape(1, H), W2, b2.reshape(1, H))


def _edge_mlp_body(xe_ref, wm1_ref, bm1_ref, g1_ref, be1_ref,
                   wm2_ref, bm2_ref, g2_ref, be2_ref, x1_ref, x2_ref):
    xe = xe_ref[...]

    def ln_relu(z, g, b):
        mu = jnp.mean(z, axis=-1, keepdims=True)
        var = jnp.mean((z - mu) ** 2, axis=-1, keepdims=True)
        return jax.nn.relu((z - mu) * jax.lax.rsqrt(var + 1e-5) * g + b)

    z1 = jax.lax.dot(xe, wm1_ref[...]) + bm1_ref[...]
    x1_ref[...] = ln_relu(z1, g1_ref[...], be1_ref[...])
    z2 = jax.lax.dot(xe, wm2_ref[...]) + bm2_ref[...]
    x2_ref[...] = ln_relu(z2, g2_ref[...], be2_ref[...])


def _tc_edge_mlps(xe, Wm1, bm1, g1, be1, Wm2, bm2, g2, be2):
    blk = 8192
    grid = E // blk
    row_spec = pl.BlockSpec((blk, H), lambda i: (i, 0))
    w_spec = pl.BlockSpec((H, H), lambda i: (0, 0))
    v_spec = pl.BlockSpec((1, H), lambda i: (0, 0))
    return pl.pallas_call(
        _edge_mlp_body,
        grid=(grid,),
        in_specs=[row_spec, w_spec, v_spec, v_spec, v_spec,
                  w_spec, v_spec, v_spec, v_spec],
        out_specs=[row_spec, row_spec],
        out_shape=[jax.ShapeDtypeStruct((E, H), jnp.float32),
                   jax.ShapeDtypeStruct((E, H), jnp.float32)],
    )(xe, Wm1, bm1.reshape(1, H), g1.reshape(1, H), be1.reshape(1, H),
      Wm2, bm2.reshape(1, H), g2.reshape(1, H), be2.reshape(1, H))


def _final_body(pv_ref, xx_ref, w3a_ref, b3a_ref, w3b_ref, b3b_ref, out_ref):
    w3a = w3a_ref[...]
    h3 = jax.nn.relu(
        jax.lax.dot(pv_ref[...], w3a[:H, :])
        + jax.lax.dot(xx_ref[...], w3a[H:, :])
        + b3a_ref[...])
    out_ref[...] = jax.lax.dot(h3, w3b_ref[...]) + b3b_ref[...]


def _tc_final(pos_val, xx, W3a, b3a, W3b, b3b):
    out2 = pl.pallas_call(
        _final_body,
        out_shape=jax.ShapeDtypeStruct((P, 1), jnp.float32),
    )(pos_val, xx, W3a, b3a.reshape(1, H), W3b, b3b.reshape(1, 1))
    return out2[:, 0]


def _fwl_body(lr_hbm, lc_hbm, x1_hbm, x2_hbm, pa_hbm, pb_hbm, out_hbm,
              pa_v, pb_v, rowbuf, colbuf, wl1, wl2, g1buf, g2buf, pvbuf,
              sem):
    wid = lax.axis_index("s") * 2 + lax.axis_index("c")
    base = wid * PP
    pltpu.sync_copy(pa_hbm.at[pl.ds(base, PP)], pa_v)
    pltpu.sync_copy(pb_hbm.at[pl.ds(base, PP)], pb_v)
    d1 = pltpu.async_copy(lr_hbm.at[pa_v], rowbuf, sem)
    d2 = pltpu.async_copy(lc_hbm.at[pb_v], colbuf, sem)
    d1.wait()
    d2.wait()

    zf = jnp.zeros((16,), jnp.float32)
    zi = jnp.zeros((16,), jnp.int32)

    def per_p(p, carry):
        def scan_slice(j, off):
            r = rowbuf[p, pl.ds(j * 16, 16)]
            c = colbuf[p, pl.ds(j * 16, 16)]
            m = (r > 0) & (c > 0)
            cnt = plsc.all_reduce_population_count(m)[0]

            @pl.when(cnt > 0)
            def _():
                plsc.store_compressed(wl1.at[pl.ds(off, 16)], r - 1, mask=m)
                plsc.store_compressed(wl2.at[pl.ds(off, 16)], c - 1, mask=m)

            return off + cnt

        npairs = lax.fori_loop(0, NSL, scan_slice, jnp.int32(0))
        wl1[pl.ds(npairs, 16)] = jnp.full((16,), E, jnp.int32)
        wl2[pl.ds(npairs, 16)] = jnp.full((16,), E, jnp.int32)
        nch = (npairs + 15) // 16

        def chunk(ch, acc):
            alo, ahi = acc
            e1 = wl1[pl.ds(ch * 16, 16)]
            e2 = wl2[pl.ds(ch * 16, 16)]
            g1 = pltpu.async_copy(x2_hbm.at[e1], g2buf, sem)
            g2 = pltpu.async_copy(x1_hbm.at[e2], g1buf, sem)
            g1.wait()
            g2.wait()
            for r in range(16):
                alo = alo + g2buf[r, pl.ds(0, 16)] * g1buf[r, pl.ds(0, 16)]
                ahi = ahi + g2buf[r, pl.ds(16, 16)] * g1buf[r, pl.ds(16, 16)]
            return (alo, ahi)

        alo, ahi = lax.fori_loop(0, nch, chunk, (zf, zf))
        pvbuf[p, pl.ds(0, 16)] = alo
        pvbuf[p, pl.ds(16, 16)] = ahi
        return carry

    lax.fori_loop(0, PP, per_p, jnp.int32(0))
    pltpu.sync_copy(pvbuf, out_hbm.at[pl.ds(base, PP)])


def _sc_fwl(Lr, Lc, x1p, x2p, pa, pb):
    f = pl.kernel(
        _fwl_body,
        out_type=jax.ShapeDtypeStruct((P, H), jnp.float32),
        mesh=plsc.VectorSubcoreMesh(core_axis_name="c", subcore_axis_name="s"),
        compiler_params=pltpu.CompilerParams(
            needs_layout_passes=False, use_tc_tiling_on_sc=False),
        scratch_types=[
            pltpu.VMEM((PP,), jnp.int32),
            pltpu.VMEM((PP,), jnp.int32),
            pltpu.VMEM((PP, N), jnp.int32),
            pltpu.VMEM((PP, N), jnp.int32),
            pltpu.VMEM((WLCAP,), jnp.int32),
            pltpu.VMEM((WLCAP,), jnp.int32),
            pltpu.VMEM((16, H), jnp.float32),
            pltpu.VMEM((16, H), jnp.float32),
            pltpu.VMEM((PP, H), jnp.float32),
            pltpu.SemaphoreType.DMA,
        ],
    )
    return f(Lr, Lc, x1p, x2p, pa, pb)


def kernel(x, ei, pos, W1, b1, W2, b2, Wm1, bm1, g1, be1, Wm2, bm2, g2, be2,
           W3a, b3a, W3b, b3b):
    src, dst = ei[0], ei[1]

    loop = jnp.arange(N)
    s_all = jnp.concatenate([src, loop])
    d_all = jnp.concatenate([dst, loop])
    deg = jnp.zeros((N,), jnp.float32).at[d_all].add(1.0)
    dinv = jnp.where(deg > 0, deg ** -0.5, 0.0)
    norm = dinv[s_all] * dinv[d_all]
    ahat = jnp.zeros((N, N), jnp.float32).at[d_all, s_all].add(norm)

    h = _tc_gcn(x, ahat, W1, b1, W2, b2)

    xx = h[pos[0]] * h[pos[1]]
    xe = h[src] * h[dst]

    x1, x2 = _tc_edge_mlps(xe, Wm1, bm1, g1, be1, Wm2, bm2, g2, be2)

    eid = jnp.arange(E, dtype=jnp.int32) + 1
    Lr = jnp.zeros((N, N), jnp.int32).at[src, dst].set(eid)
    Lc = jnp.zeros((N, N), jnp.int32).at[dst, src].set(eid)
    zpad = jnp.zeros((8, H), jnp.float32)
    x1p = jnp.concatenate([x1, zpad])
    x2p = jnp.concatenate([x2, zpad])
    pos_val = _sc_fwl(Lr, Lc, x1p, x2p,
                      pos[0].astype(jnp.int32), pos[1].astype(jnp.int32))

    return _tc_final(pos_val, xx, W3a, b3a, W3b, b3b)


# speedup vs baseline: 8.5687x; 1.1996x over previous
import functools

import jax
import jax.numpy as jnp
from jax import lax
from jax.experimental import pallas as pl
from jax.experimental.pallas import tpu as pltpu
from jax.experimental.pallas import tpu_sc as plsc

N = 1024
E = 32768
P = 768
F = 128
H = 32

NW = 32
PP = P // NW
NSL = N // 16
WLCAP = N + 16


def _gcn_body(x_ref, ahat_ref, w1_ref, b1_ref, w2_ref, b2_ref, h_ref):
    hp = jax.lax.Precision.HIGHEST
    xw1 = jax.lax.dot(x_ref[...], w1_ref[...])
    h1 = jax.lax.dot(ahat_ref[...], xw1, precision=hp) + b1_ref[...]
    h1w2 = jax.lax.dot(h1, w2_ref[...])
    h_ref[...] = jax.lax.dot(ahat_ref[...], h1w2, precision=hp) + b2_ref[...]


def _tc_gcn(x, ahat, W1, b1, W2, b2):
    return pl.pallas_call(
        _gcn_body,
        out_shape=jax.ShapeDtypeStruct((N, H), jnp.float32),
    )(x, ahat, W1, b1.reshape(1, H), W2, b2.reshape(1, H))


def _edge_mlp_body(hs_ref, hd_ref, wm1_ref, bm1_ref, g1_ref, be1_ref,
                   wm2_ref, bm2_ref, g2_ref, be2_ref, x1_ref, x2_ref):
    xe = hs_ref[...] * hd_ref[...]

    def ln_relu(z, g, b):
        mu = jnp.mean(z, axis=-1, keepdims=True)
        var = jnp.mean((z - mu) ** 2, axis=-1, keepdims=True)
        return jax.nn.relu((z - mu) * jax.lax.rsqrt(var + 1e-5) * g + b)

    z1 = jax.lax.dot(xe, wm1_ref[...]) + bm1_ref[...]
    x1_ref[...] = ln_relu(z1, g1_ref[...], be1_ref[...])
    z2 = jax.lax.dot(xe, wm2_ref[...]) + bm2_ref[...]
    x2_ref[...] = ln_relu(z2, g2_ref[...], be2_ref[...])


def _tc_edge_mlps(hs, hd, Wm1, bm1, g1, be1, Wm2, bm2, g2, be2):
    blk = 8192
    grid = E // blk
    row_spec = pl.BlockSpec((blk, H), lambda i: (i, 0))
    w_spec = pl.BlockSpec((H, H), lambda i: (0, 0))
    v_spec = pl.BlockSpec((1, H), lambda i: (0, 0))
    return pl.pallas_call(
        _edge_mlp_body,
        grid=(grid,),
        in_specs=[row_spec, row_spec, w_spec, v_spec, v_spec, v_spec,
                  w_spec, v_spec, v_spec, v_spec],
        out_specs=[row_spec, row_spec],
        out_shape=[jax.ShapeDtypeStruct((E, H), jnp.float32),
                   jax.ShapeDtypeStruct((E, H), jnp.float32)],
    )(hs, hd, Wm1, bm1.reshape(1, H), g1.reshape(1, H), be1.reshape(1, H),
      Wm2, bm2.reshape(1, H), g2.reshape(1, H), be2.reshape(1, H))


def _final_body(pv_ref, ha_ref, hb_ref, w3a_ref, b3a_ref, w3b_ref, b3b_ref,
                out_ref):
    w3a = w3a_ref[...]
    h3 = jax.nn.relu(
        jax.lax.dot(pv_ref[...], w3a[:H, :])
        + jax.lax.dot(ha_ref[...] * hb_ref[...], w3a[H:, :])
        + b3a_ref[...])
    out_ref[...] = jax.lax.dot(h3, w3b_ref[...]) + b3b_ref[...]


def _tc_final(pos_val, ha, hb, W3a, b3a, W3b, b3b):
    out2 = pl.pallas_call(
        _final_body,
        out_shape=jax.ShapeDtypeStruct((P, 1), jnp.float32),
    )(pos_val, ha, hb, W3a, b3a.reshape(1, H), W3b, b3b.reshape(1, 1))
    return out2[:, 0]


def _fwl_body(lr_hbm, lc_hbm, x1_hbm, x2_hbm, pa_hbm, pb_hbm, out_hbm,
              pa_v, pb_v, rowbuf, colbuf, wl1, wl2, g1buf, g2buf, pvbuf,
              sem):
    wid = lax.axis_index("s") * 2 + lax.axis_index("c")
    base = wid * PP
    pltpu.sync_copy(pa_hbm.at[pl.ds(base, PP)], pa_v)
    pltpu.sync_copy(pb_hbm.at[pl.ds(base, PP)], pb_v)
    d1 = pltpu.async_copy(lr_hbm.at[pa_v], rowbuf, sem)
    d2 = pltpu.async_copy(lc_hbm.at[pb_v], colbuf, sem)
    d1.wait()
    d2.wait()

    zf = jnp.zeros((16,), jnp.float32)
    zi = jnp.zeros((16,), jnp.int32)

    def per_p(p, carry):
        def scan_slice(j, off):
            r = rowbuf[p, pl.ds(j * 16, 16)]
            c = colbuf[p, pl.ds(j * 16, 16)]
            m = (r > 0) & (c > 0)
            cnt = plsc.all_reduce_population_count(m)[0]

            @pl.when(cnt > 0)
            def _():
                plsc.store_compressed(wl1.at[pl.ds(off, 16)], r - 1, mask=m)
                plsc.store_compressed(wl2.at[pl.ds(off, 16)], c - 1, mask=m)

            return off + cnt

        npairs = lax.fori_loop(0, NSL, scan_slice, jnp.int32(0))
        wl1[pl.ds(npairs, 16)] = jnp.full((16,), E, jnp.int32)
        wl2[pl.ds(npairs, 16)] = jnp.full((16,), E, jnp.int32)
        nch = (npairs + 15) // 16

        def chunk(ch, acc):
            alo, ahi = acc
            e1 = wl1[pl.ds(ch * 16, 16)]
            e2 = wl2[pl.ds(ch * 16, 16)]
            g1 = pltpu.async_copy(x2_hbm.at[e1], g2buf, sem)
            g2 = pltpu.async_copy(x1_hbm.at[e2], g1buf, sem)
            g1.wait()
            g2.wait()
            for r in range(16):
                alo = alo + g2buf[r, pl.ds(0, 16)] * g1buf[r, pl.ds(0, 16)]
                ahi = ahi + g2buf[r, pl.ds(16, 16)] * g1buf[r, pl.ds(16, 16)]
            return (alo, ahi)

        alo, ahi = lax.fori_loop(0, nch, chunk, (zf, zf))
        pvbuf[p, pl.ds(0, 16)] = alo
        pvbuf[p, pl.ds(16, 16)] = ahi
        return carry

    lax.fori_loop(0, PP, per_p, jnp.int32(0))
    pltpu.sync_copy(pvbuf, out_hbm.at[pl.ds(base, PP)])


def _sc_fwl(Lr, Lc, x1p, x2p, pa, pb):
    f = pl.kernel(
        _fwl_body,
        out_type=jax.ShapeDtypeStruct((P, H), jnp.float32),
        mesh=plsc.VectorSubcoreMesh(core_axis_name="c", subcore_axis_name="s"),
        compiler_params=pltpu.CompilerParams(
            needs_layout_passes=False, use_tc_tiling_on_sc=False),
        scratch_types=[
            pltpu.VMEM((PP,), jnp.int32),
            pltpu.VMEM((PP,), jnp.int32),
            pltpu.VMEM((PP, N), jnp.int32),
            pltpu.VMEM((PP, N), jnp.int32),
            pltpu.VMEM((WLCAP,), jnp.int32),
            pltpu.VMEM((WLCAP,), jnp.int32),
            pltpu.VMEM((16, H), jnp.float32),
            pltpu.VMEM((16, H), jnp.float32),
            pltpu.VMEM((PP, H), jnp.float32),
            pltpu.SemaphoreType.DMA,
        ],
    )
    return f(Lr, Lc, x1p, x2p, pa, pb)


EPW = E // NW
PPW = P // NW


def _gath_body(h_hbm, src_hbm, dst_hbm, pa_hbm, pb_hbm,
               hs_hbm, hd_hbm, ha_hbm, hb_hbm,
               iv1, iv2, rows1, rows2, ivp1, ivp2, rp1, rp2, sem1, sem2):
    wid = lax.axis_index("s") * 2 + lax.axis_index("c")
    eb = wid * EPW
    pb_ = wid * PPW
    pltpu.sync_copy(src_hbm.at[pl.ds(eb, EPW)], iv1)
    pltpu.sync_copy(dst_hbm.at[pl.ds(eb, EPW)], iv2)
    pltpu.sync_copy(pa_hbm.at[pl.ds(pb_, PPW)], ivp1)
    pltpu.sync_copy(pb_hbm.at[pl.ds(pb_, PPW)], ivp2)
    a1 = pltpu.async_copy(h_hbm.at[iv1], rows1, sem1)
    a2 = pltpu.async_copy(h_hbm.at[iv2], rows2, sem2)
    a1.wait()
    b1 = pltpu.async_copy(h_hbm.at[ivp1], rp1, sem1)
    pltpu.sync_copy(rows1, hs_hbm.at[pl.ds(eb, EPW)])
    a2.wait()
    b2 = pltpu.async_copy(h_hbm.at[ivp2], rp2, sem2)
    pltpu.sync_copy(rows2, hd_hbm.at[pl.ds(eb, EPW)])
    b1.wait()
    pltpu.sync_copy(rp1, ha_hbm.at[pl.ds(pb_, PPW)])
    b2.wait()
    pltpu.sync_copy(rp2, hb_hbm.at[pl.ds(pb_, PPW)])


def _sc_gather(h, src, dst, pa, pb):
    f = pl.kernel(
        _gath_body,
        out_type=(jax.ShapeDtypeStruct((E, H), jnp.float32),
                  jax.ShapeDtypeStruct((E, H), jnp.float32),
                  jax.ShapeDtypeStruct((P, H), jnp.float32),
                  jax.ShapeDtypeStruct((P, H), jnp.float32)),
        mesh=plsc.VectorSubcoreMesh(core_axis_name="c", subcore_axis_name="s"),
        compiler_params=pltpu.CompilerParams(
            needs_layout_passes=False, use_tc_tiling_on_sc=False),
        scratch_types=[
            pltpu.VMEM((EPW,), jnp.int32),
            pltpu.VMEM((EPW,), jnp.int32),
            pltpu.VMEM((EPW, H), jnp.float32),
            pltpu.VMEM((EPW, H), jnp.float32),
            pltpu.VMEM((PPW,), jnp.int32),
            pltpu.VMEM((PPW,), jnp.int32),
            pltpu.VMEM((PPW, H), jnp.float32),
            pltpu.VMEM((PPW, H), jnp.float32),
            pltpu.SemaphoreType.DMA,
            pltpu.SemaphoreType.DMA,
        ],
    )
    return f(h, src, dst, pa, pb)


def kernel(x, ei, pos, W1, b1, W2, b2, Wm1, bm1, g1, be1, Wm2, bm2, g2, be2,
           W3a, b3a, W3b, b3b):
    src, dst = ei[0], ei[1]

    loop = jnp.arange(N)
    s_all = jnp.concatenate([src, loop])
    d_all = jnp.concatenate([dst, loop])
    deg = jnp.zeros((N,), jnp.float32).at[d_all].add(1.0)
    dinv = jnp.where(deg > 0, deg ** -0.5, 0.0)
    norm = dinv[s_all] * dinv[d_all]
    ahat = jnp.zeros((N, N), jnp.float32).at[d_all, s_all].add(norm)

    h = _tc_gcn(x, ahat, W1, b1, W2, b2)

    pa = pos[0].astype(jnp.int32)
    pb = pos[1].astype(jnp.int32)
    hs, hd, ha, hb = _sc_gather(h, src.astype(jnp.int32),
                                dst.astype(jnp.int32), pa, pb)

    x1, x2 = _tc_edge_mlps(hs, hd, Wm1, bm1, g1, be1, Wm2, bm2, g2, be2)

    eid = jnp.arange(E, dtype=jnp.int32) + 1
    Lr = jnp.zeros((N, N), jnp.int32).at[src, dst].set(eid)
    Lc = jnp.zeros((N, N), jnp.int32).at[dst, src].set(eid)
    zpad = jnp.zeros((8, H), jnp.float32)
    x1p = jnp.concatenate([x1, zpad])
    x2p = jnp.concatenate([x2, zpad])
    pos_val = _sc_fwl(Lr, Lc, x1p, x2p, pa, pb)

    return _tc_final(pos_val, ha, hb, W3a, b3a, W3b, b3b)
